# baseline (speedup 1.0000x reference)
import functools

import jax
import jax.numpy as jnp
from jax import lax
from jax.experimental import pallas as pl
from jax.experimental.pallas import tpu as pltpu
from jax.experimental.pallas import tpu_sc as plsc


_ROWS = 4 * 4096
_K = 128
_C = 2048
_STRIDE = 16
_L = 16
_NC = 2
_NS = 16
_NW = _NC * _NS
_RPW = _ROWS // _NW
_CH = 32
_NCHUNK = _RPW // _CH


def _sc_body(x_hbm, out_hbm, in_v, out_v):
    wid = lax.axis_index("s") * _NC + lax.axis_index("c")
    row0 = wid * _RPW
    lanes16 = lax.iota(jnp.int32, _L) * _STRIDE

    def zero_body(i, _):
        for t in range(8):
            out_v[pl.ds((i * 8 + t) * _L, _L)] = jnp.zeros((_L,), jnp.float32)
        return 0
    lax.fori_loop(0, _CH * _C // _L // 8, zero_body, 0)

    def chunk_body(g, _):
        rbase = row0 + g * _CH
        pltpu.sync_copy(x_hbm.at[pl.ds(rbase * _K, _CH * _K)], in_v)

        def scat_body(j, _):
            for t in range(8):
                jj = j * 8 + t
                vec = in_v[pl.ds(jj * _L, _L)]
                idx = jj * (_L * _STRIDE) + lanes16
                plsc.store_scatter(out_v, [idx], vec)
            return 0
        lax.fori_loop(0, _CH * _K // _L // 8, scat_body, 0)

        pltpu.sync_copy(out_v, out_hbm.at[pl.ds(rbase * _C, _CH * _C)])
        return 0
    lax.fori_loop(0, _NCHUNK, chunk_body, 0)


@functools.partial(jax.jit, static_argnames=())
def _sc_upsample(xf):
    mesh = plsc.VectorSubcoreMesh(core_axis_name="c", subcore_axis_name="s")
    fn = pl.kernel(
        _sc_body,
        out_type=jax.ShapeDtypeStruct((_ROWS * _C,), jnp.float32),
        mesh=mesh,
        scratch_types=[
            pltpu.VMEM((_CH * _K,), jnp.float32),
            pltpu.VMEM((_CH * _C,), jnp.float32),
        ],
        compiler_params=pltpu.CompilerParams(needs_layout_passes=False),
    )
    return fn(xf)


def kernel(x, attention_output):
    del attention_output
    xf = x.reshape(_ROWS * _K)
    return _sc_upsample(xf).reshape(4, 4096, _C)


# speedup vs baseline: 2.3504x; 2.3504x over previous
import functools

import jax
import jax.numpy as jnp
from jax import lax
from jax.experimental import pallas as pl
from jax.experimental.pallas import tpu as pltpu
from jax.experimental.pallas import tpu_sc as plsc


_ROWS = 4 * 4096
_K = 128
_C = 2048
_STRIDE = 16
_L = 16
_NC = 2
_NS = 16
_NW = _NC * _NS
_RPW = _ROWS // _NW
_CH = 32
_NCHUNK = _RPW // _CH


def _sc_body(x_hbm, out_hbm, in_v, out_v):
    wid = lax.axis_index("s") * _NC + lax.axis_index("c")
    row0 = wid * _RPW
    lanes16 = lax.iota(jnp.int32, _L) * _STRIDE

    def zero_body(i, _):
        r = i // (_C // _L // 8)
        c0 = (i % (_C // _L // 8)) * (_L * 8)
        for t in range(8):
            out_v[r, pl.ds(c0 + t * _L, _L)] = jnp.zeros((_L,), jnp.float32)
        return 0
    lax.fori_loop(0, _CH * _C // _L // 8, zero_body, 0)

    def chunk_body(g, _):
        rbase = row0 + g * _CH
        pltpu.sync_copy(x_hbm.at[pl.ds(rbase, _CH), :], in_v)

        def scat_body(r, _):
            rvec = jnp.full((_L,), 0, jnp.int32) + r
            for j in range(_K // _L):
                vec = in_v[r, pl.ds(j * _L, _L)]
                cvec = j * (_L * _STRIDE) + lanes16
                plsc.store_scatter(out_v, [rvec, cvec], vec)
            return 0
        lax.fori_loop(0, _CH, scat_body, 0)

        pltpu.sync_copy(out_v, out_hbm.at[pl.ds(rbase, _CH), :])
        return 0
    lax.fori_loop(0, _NCHUNK, chunk_body, 0)


@jax.jit
def _sc_upsample(xf):
    mesh = plsc.VectorSubcoreMesh(core_axis_name="c", subcore_axis_name="s")
    fn = pl.kernel(
        _sc_body,
        out_type=jax.ShapeDtypeStruct((_ROWS, _C), jnp.float32),
        mesh=mesh,
        scratch_types=[
            pltpu.VMEM((_CH, _K), jnp.float32),
            pltpu.VMEM((_CH, _C), jnp.float32),
        ],
        compiler_params=pltpu.CompilerParams(
            needs_layout_passes=False,
            use_tc_tiling_on_sc=True,
        ),
    )
    return fn(xf)


def kernel(x, attention_output):
    del attention_output
    xf = x.reshape(_ROWS, _K)
    return _sc_upsample(xf).reshape(4, 4096, _C)


# speedup vs baseline: 2.8400x; 1.2083x over previous
import jax
import jax.numpy as jnp
from jax import lax
from jax.experimental import pallas as pl
from jax.experimental.pallas import tpu as pltpu
from jax.experimental.pallas import tpu_sc as plsc


_ROWS = 4 * 4096
_K = 128
_C = 2048
_STRIDE = 16
_L = 16
_NC = 2
_NS = 16
_NW = _NC * _NS
_RPW = _ROWS // _NW
_CH = 16
_NB = 2
_NPAIR = _RPW // (_CH * _NB)


def _sc_body(x_hbm, out_hbm, in_v0, in_v1, out_v0, out_v1, sem0, sem1):
    wid = lax.axis_index("s") * _NC + lax.axis_index("c")
    row0 = wid * _RPW
    lanes16 = lax.iota(jnp.int32, _L) * _STRIDE
    in_v = (in_v0, in_v1)
    out_v = (out_v0, out_v1)
    sem = (sem0, sem1)

    def zero_body(i, _):
        r = i // (_C // _L // 8)
        c0 = (i % (_C // _L // 8)) * (_L * 8)
        for b in range(_NB):
            for t in range(8):
                out_v[b][r, pl.ds(c0 + t * _L, _L)] = (
                    jnp.zeros((_L,), jnp.float32))
        return 0
    lax.fori_loop(0, _CH * _C // _L // 8, zero_body, 0)

    def pair_body(g2, _):
        for b in range(_NB):
            g = g2 * _NB + b
            rbase = row0 + g * _CH

            @pl.when(g2 > 0)
            def _():
                pltpu.make_async_copy(
                    out_v[b], out_hbm.at[pl.ds(row0, _CH), :], sem[b]).wait()

            pltpu.sync_copy(x_hbm.at[pl.ds(rbase, _CH), :], in_v[b])

            def scat_body(r, _, b=b):
                rvec = jnp.full((_L,), 0, jnp.int32) + r
                for j in range(_K // _L):
                    vec = in_v[b][r, pl.ds(j * _L, _L)]
                    cvec = j * (_L * _STRIDE) + lanes16
                    plsc.store_scatter(out_v[b], [rvec, cvec], vec)
                return 0
            lax.fori_loop(0, _CH, scat_body, 0)

            pltpu.async_copy(out_v[b], out_hbm.at[pl.ds(rbase, _CH), :],
                             sem[b])
        return 0
    lax.fori_loop(0, _NPAIR, pair_body, 0)

    for b in range(_NB):
        pltpu.make_async_copy(
            out_v[b], out_hbm.at[pl.ds(row0, _CH), :], sem[b]).wait()


@jax.jit
def _sc_upsample(xf):
    mesh = plsc.VectorSubcoreMesh(core_axis_name="c", subcore_axis_name="s")
    fn = pl.kernel(
        _sc_body,
        out_type=jax.ShapeDtypeStruct((_ROWS, _C), jnp.float32),
        mesh=mesh,
        scratch_types=[
            pltpu.VMEM((_CH, _K), jnp.float32),
            pltpu.VMEM((_CH, _K), jnp.float32),
            pltpu.VMEM((_CH, _C), jnp.float32),
            pltpu.VMEM((_CH, _C), jnp.float32),
            pltpu.SemaphoreType.DMA,
            pltpu.SemaphoreType.DMA,
        ],
        compiler_params=pltpu.CompilerParams(
            needs_layout_passes=False,
            use_tc_tiling_on_sc=True,
        ),
    )
    return fn(xf)


def kernel(x, attention_output):
    del attention_output
    xf = x.reshape(_ROWS, _K)
    return _sc_upsample(xf).reshape(4, 4096, _C)


# speedup vs baseline: 2.8401x; 1.0000x over previous
import jax
import jax.numpy as jnp
from jax import lax
from jax.experimental import pallas as pl
from jax.experimental.pallas import tpu as pltpu
from jax.experimental.pallas import tpu_sc as plsc


_ROWS = 4 * 4096
_K = 128
_C = 2048
_STRIDE = 16
_L = 16
_NC = 2
_NS = 16
_NW = _NC * _NS
_RPW = _ROWS // _NW
_CH = 16
_NB = 2
_NPAIR = _RPW // (_CH * _NB)


def _sc_body(x_hbm, out_hbm, in_v0, in_v1, out_v0, out_v1, sem0, sem1):
    wid = lax.axis_index("s") * _NC + lax.axis_index("c")
    row0 = wid * _RPW
    lanes16 = lax.iota(jnp.int32, _L) * _STRIDE
    in_v = (in_v0, in_v1)
    out_v = (out_v0, out_v1)
    sem = (sem0, sem1)

    def zero_body(i, _):
        r = i // (_C // _L // 8)
        c0 = (i % (_C // _L // 8)) * (_L * 8)
        for b in range(_NB):
            for t in range(8):
                out_v[b][r, pl.ds(c0 + t * _L, _L)] = (
                    jnp.zeros((_L,), jnp.float32))
        return 0
    lax.fori_loop(0, _CH * _C // _L // 8, zero_body, 0)

    def pair_body(g2, _):
        for b in range(_NB):
            g = g2 * _NB + b
            rbase = row0 + g * _CH

            @pl.when(g2 > 0)
            def _():
                pltpu.make_async_copy(
                    out_v[b], out_hbm.at[pl.ds(row0, _CH), :], sem[b]).wait()

            pltpu.sync_copy(x_hbm.at[pl.ds(rbase, _CH), :], in_v[b])

            def scat_body(r, _, b=b):
                rvec = jnp.full((_L,), 0, jnp.int32) + r
                for j in range(_K // _L):
                    vec = in_v[b][r, pl.ds(j * _L, _L)]
                    cvec = j * (_L * _STRIDE) + lanes16
                    plsc.store_scatter(out_v[b], [rvec, cvec], vec)
                return 0
            lax.fori_loop(0, _CH, scat_body, 0)

            pltpu.async_copy(out_v[b], out_hbm.at[pl.ds(rbase, _CH), :],
                             sem[b])
        return 0
    lax.fori_loop(0, _NPAIR, pair_body, 0)

    for b in range(_NB):
        pltpu.make_async_copy(
            out_v[b], out_hbm.at[pl.ds(row0, _CH), :], sem[b]).wait()


@jax.jit
def _sc_upsample(xf):
    mesh = plsc.VectorSubcoreMesh(core_axis_name="c", subcore_axis_name="s")
    fn = pl.kernel(
        _sc_body,
        out_type=jax.ShapeDtypeStruct((_ROWS, _C), jnp.float32),
        mesh=mesh,
        scratch_types=[
            pltpu.VMEM((_CH, _K), jnp.float32),
            pltpu.VMEM((_CH, _K), jnp.float32),
            pltpu.VMEM((_CH, _C), jnp.float32),
            pltpu.VMEM((_CH, _C), jnp.float32),
            pltpu.SemaphoreType.DMA,
            pltpu.SemaphoreType.DMA,
        ],
        compiler_params=pltpu.CompilerParams(
            needs_layout_passes=False,
            use_tc_tiling_on_sc=True,
            disable_bounds_checks=True,
            skip_device_barrier=True,
        ),
    )
    return fn(xf)


def kernel(x, attention_output):
    del attention_output
    xf = x.reshape(_ROWS, _K)
    return _sc_upsample(xf).reshape(4, 4096, _C)


# speedup vs baseline: 2.8628x; 1.0080x over previous
import jax
import jax.numpy as jnp
from jax import lax
from jax.experimental import pallas as pl
from jax.experimental.pallas import tpu as pltpu
from jax.experimental.pallas import tpu_sc as plsc


_ROWS = 4 * 4096
_K = 128
_C = 2048
_STRIDE = 16
_L = 16
_NC = 2
_NS = 16
_NW = _NC * _NS
_RPW = _ROWS // _NW
_CH = 16
_NB = 2
_NPAIR = _RPW // (_CH * _NB)


def _sc_body(x_hbm, out_hbm, in_v0, in_v1, out_v0, out_v1, sem0, sem1):
    wid = lax.axis_index("s") * _NC + lax.axis_index("c")
    row0 = wid * _RPW
    lanes16 = lax.iota(jnp.int32, _L) * _STRIDE
    in_v = (in_v0, in_v1)
    out_v = (out_v0, out_v1)
    sem = (sem0, sem1)

    def zero_body(i, _):
        r = i // (_C // _L // 8)
        c0 = (i % (_C // _L // 8)) * (_L * 8)
        for b in range(_NB):
            for t in range(8):
                out_v[b][r, pl.ds(c0 + t * _L, _L)] = (
                    jnp.zeros((_L,), jnp.float32))
        return 0
    lax.fori_loop(0, _CH * _C // _L // 8, zero_body, 0)

    def pair_body(g2, _):
        for b in range(_NB):
            g = g2 * _NB + b
            rbase = row0 + g * _CH

            @pl.when(g2 > 0)
            def _():
                pltpu.make_async_copy(
                    out_v[b], out_hbm.at[pl.ds(row0, _CH), :], sem[b]).wait()

            pltpu.sync_copy(x_hbm.at[pl.ds(rbase, _CH), :], in_v[b])

            def scat_body(r, _, b=b):
                rvec = jnp.full((_L,), 0, jnp.int32) + r
                for j in range(_K // _L):
                    vec = in_v[b][r, pl.ds(j * _L, _L)]
                    cvec = j * (_L * _STRIDE) + lanes16
                    plsc.store_scatter(out_v[b], [rvec, cvec], vec)
                return 0
            lax.fori_loop(0, _CH, scat_body, 0)

            pltpu.async_copy(out_v[b], out_hbm.at[pl.ds(rbase, _CH), :],
                             sem[b])
        return 0
    lax.fori_loop(0, _NPAIR, pair_body, 0)

    for b in range(_NB):
        pltpu.make_async_copy(
            out_v[b], out_hbm.at[pl.ds(row0, _CH), :], sem[b]).wait()


@jax.jit
def _sc_upsample(xf):
    mesh = plsc.VectorSubcoreMesh(core_axis_name="c", subcore_axis_name="s")
    fn = pl.kernel(
        _sc_body,
        out_type=jax.ShapeDtypeStruct((_ROWS, _C), jnp.float32),
        mesh=mesh,
        scratch_types=[
            pltpu.VMEM((_CH, _K), jnp.float32),
            pltpu.VMEM((_CH, _K), jnp.float32),
            pltpu.VMEM((_CH, _C), jnp.float32),
            pltpu.VMEM((_CH, _C), jnp.float32),
            pltpu.SemaphoreType.DMA,
            pltpu.SemaphoreType.DMA,
        ],
        compiler_params=pltpu.CompilerParams(
            needs_layout_passes=False,
            use_tc_tiling_on_sc=True,
        ),
    )
    return fn(xf)


def kernel(x, attention_output):
    del attention_output
    xf = x.reshape(_ROWS, _K)
    return _sc_upsample(xf).reshape(4, 4096, _C)
